# baseline (speedup 1.0000x reference)
import functools

import jax
import jax.numpy as jnp
from jax import lax
from jax.experimental import pallas as pl
from jax.experimental.pallas import tpu as pltpu
from jax.experimental.pallas import tpu_sc as plsc

N = 10000
E = 320000
NC, NS, LANES = 2, 16, 16
NW = NC * NS
BLK = 128
NPAD = 10112
RPT = NPAD // NS
DUMP = N

f32 = jnp.float32
i32 = jnp.int32

_MESH = plsc.VectorSubcoreMesh(core_axis_name="c", subcore_axis_name="s")


def _nb(m):
    nb = -(-(m // NW) // BLK)
    return -(-nb // 8) * 8


def _pad_blocks(idx, m):
    nb = _nb(m)
    tot = NW * nb * BLK
    idx = jnp.pad(idx.astype(i32), (0, tot - m), constant_values=DUMP)
    return idx.reshape(NW * nb, BLK)


@functools.cache
def _scatter_kernel(m, d, streams):
    nb = _nb(m)
    nvb_tot = m // BLK
    two = streams == 2
    tbl_shape = (NPAD,) if d == 1 else (NPAD, d)
    vshape = (BLK,) if d == 1 else (BLK, d)
    out_sh = (NC * NPAD,) if d == 1 else (NC, NPAD, d)

    zshape = (RPT,) if d == 1 else (RPT, d)
    scratch = [pltpu.VMEM((nb, BLK), i32)] * streams + [
        pltpu.VMEM(vshape, f32),
        pltpu.VMEM(vshape, f32),
        pltpu.VMEM(zshape, f32),
        pltpu.VMEM_SHARED(tbl_shape, f32),
        pltpu.SemaphoreType.DMA,
        pltpu.SemaphoreType.DMA,
    ]

    @functools.partial(
        pl.kernel,
        out_type=jax.ShapeDtypeStruct(out_sh, f32),
        mesh=_MESH,
        scratch_types=scratch,
    )
    def k(*refs):
        if two:
            (vals_h, ia_h, ib_h, z_h, out_h, ia_v, ib_v,
             v0, v1, zbuf, tbl, sem0, sem1) = refs
        else:
            vals_h, ia_h, z_h, out_h, ia_v, v0, v1, zbuf, tbl, sem0, sem1 = refs
            ib_h = ib_v = None
        bufs = (v0, v1)
        sems = (sem0, sem1)
        c = lax.axis_index("c")
        s = lax.axis_index("s")
        w = s * NC + c
        pltpu.sync_copy(z_h, zbuf)
        pltpu.sync_copy(zbuf, tbl.at[pl.ds(s * RPT, RPT)])
        pltpu.sync_copy(ia_h.at[pl.ds(w * nb, nb)], ia_v)
        if two:
            pltpu.sync_copy(ib_h.at[pl.ds(w * nb, nb)], ib_v)
        plsc.subcore_barrier()
        nvb = jnp.minimum(nb, jnp.maximum(nvb_tot - w * nb, 0))

        for k2 in range(2):
            @pl.when(k2 < nvb)
            def _():
                base = (w * nb + k2) * BLK
                pltpu.async_copy(vals_h.at[pl.ds(base, BLK)], bufs[k2], sems[k2])

        def pairblk(j2, carry):
            for k2 in range(2):
                j = j2 * 2 + k2

                @pl.when(j < nvb)
                def _():
                    base = (w * nb + j) * BLK
                    pltpu.make_async_copy(vals_h.at[pl.ds(base, BLK)],
                                          bufs[k2], sems[k2]).wait()
                    pltpu.sync_copy(bufs[k2], tbl.at[ia_v.at[j]], add=True)
                    if two:
                        pltpu.sync_copy(bufs[k2], tbl.at[ib_v.at[j]], add=True)

                    @pl.when(j + 2 < nvb)
                    def _():
                        base2 = (w * nb + j + 2) * BLK
                        pltpu.async_copy(vals_h.at[pl.ds(base2, BLK)],
                                         bufs[k2], sems[k2])
            return carry

        lax.fori_loop(0, (nvb + 1) // 2, pairblk, 0)
        plsc.subcore_barrier()
        pltpu.sync_copy(tbl.at[pl.ds(s * RPT, RPT)], zbuf)
        if d == 1:
            pltpu.sync_copy(zbuf, out_h.at[pl.ds(c * NPAD + s * RPT, RPT)])
        else:
            pltpu.sync_copy(zbuf, out_h.at[c, pl.ds(s * RPT, RPT)])

    return k


QN = 2500
QPAD = 2560
QRPT = QPAD // NS
NDUMP = 32


@functools.cache
def _scatter_dual_kernel():
    d = 128
    cpw = E // NS
    nb = -(-(-(-cpw // BLK)) // 8) * 8
    nvb_tot = E // BLK

    scratch = [
        pltpu.VMEM((nb, BLK), i32),
        pltpu.VMEM((nb, BLK), i32),
        pltpu.VMEM((BLK, d), f32),
        pltpu.VMEM((BLK, d), f32),
        pltpu.VMEM((QRPT, d), f32),
        pltpu.VMEM_SHARED((QPAD, d), f32),
        pltpu.SemaphoreType.DMA,
        pltpu.SemaphoreType.DMA,
        pltpu.SemaphoreType.DMA,
        pltpu.SemaphoreType.DMA,
    ]

    @functools.partial(
        pl.kernel,
        out_type=(jax.ShapeDtypeStruct((4 * QPAD, d), f32),
                  jax.ShapeDtypeStruct((4 * QPAD, d), f32)),
        mesh=_MESH,
        scratch_types=scratch,
    )
    def k(va_h, ia_h, vb_h, ib1_h, ib2_h, z_h, outa_h, outb_h,
          i1_v, i2_v, v0, v1, zbuf, tbl, sem0, sem1, ssc0, ssc1):
        c = lax.axis_index("c")
        s = lax.axis_index("s")
        nvb = jnp.minimum(nb, jnp.maximum(nvb_tot - s * nb, 0))
        bufs = (v0, v1)
        sems = (sem0, sem1)
        ssems = (ssc0, ssc1)
        pltpu.sync_copy(z_h, zbuf)

        def sweep(vals_h, idx_list):
            for k2 in range(2):
                @pl.when(k2 < nvb)
                def _():
                    base = (s * nb + k2) * BLK
                    pltpu.async_copy(vals_h.at[pl.ds(base, BLK)],
                                     bufs[k2], sems[k2])

            def pair(j2, carry):
                for k2 in range(2):
                    j = j2 * 2 + k2

                    @pl.when(j < nvb)
                    def _():
                        base = (s * nb + j) * BLK
                        pltpu.make_async_copy(vals_h.at[pl.ds(base, BLK)],
                                              bufs[k2], sems[k2]).wait()
                        for iv in idx_list:
                            pltpu.async_copy(bufs[k2], tbl.at[iv.at[j]],
                                             ssems[k2], add=True)

                        @pl.when(j + 2 < nvb)
                        def _():
                            for iv in idx_list:
                                pltpu.make_async_copy(
                                    bufs[k2], tbl.at[iv.at[j]],
                                    ssems[k2]).wait()
                            base2 = (s * nb + j + 2) * BLK
                            pltpu.async_copy(vals_h.at[pl.ds(base2, BLK)],
                                             bufs[k2], sems[k2])

                        @pl.when(j + 2 >= nvb)
                        def _():
                            for iv in idx_list:
                                pltpu.make_async_copy(
                                    bufs[k2], tbl.at[iv.at[j]],
                                    ssems[k2]).wait()
                return carry

            lax.fori_loop(0, (nvb + 1) // 2, pair, 0)

        def run_pass(q, vals_h, idx_hs, idx_vs, out_h):
            pltpu.sync_copy(zbuf, tbl.at[pl.ds(s * QRPT, QRPT)])
            for ih, iv in zip(idx_hs, idx_vs):
                pltpu.sync_copy(ih.at[q, pl.ds(s * nb, nb)], iv)
            plsc.subcore_barrier()
            sweep(vals_h, idx_vs)
            plsc.subcore_barrier()
            pltpu.sync_copy(tbl.at[pl.ds(s * QRPT, QRPT)], zbuf)
            pltpu.sync_copy(zbuf, out_h.at[pl.ds(q * QPAD + s * QRPT, QRPT)])
            plsc.subcore_barrier()
            pltpu.sync_copy(z_h, zbuf)

        for p in range(2):
            run_pass(p * NC + c, va_h, [ia_h], [i1_v], outa_h)
        for p in range(2):
            run_pass(p * NC + c, vb_h, [ib1_h, ib2_h], [i1_v, i2_v], outb_h)

    return k


def _ns_idx(idx, valid=None):
    e_iota = jnp.arange(E, dtype=i32)
    dump = QN + (e_iota & (NDUMP - 1))
    outs = []
    for q in range(4):
        loc = idx - q * QN
        ok = (idx >= q * QN) & (idx < (q + 1) * QN)
        if valid is not None:
            ok = ok & valid
        outs.append(jnp.where(ok, loc, dump))
    return _pad_blocks_ns(jnp.stack(outs))


def _pad_blocks_ns(idx4):
    cpw = E // NS
    nb = -(-(-(-cpw // BLK)) // 8) * 8
    tot = NS * nb * BLK
    idx4 = jnp.pad(idx4, ((0, 0), (0, tot - E)), constant_values=QN)
    return idx4.reshape(4, NS * nb, BLK)


@functools.cache
def _gather_kernel(m, d, streams):
    nb = _nb(m)
    nvb_tot = m // BLK
    two = streams == 2
    via_spmem = d < 128
    rshape = (BLK,) if d == 1 else (BLK, d)
    out_sh = (m,) if d == 1 else (m, d)
    tbl_shape = (NPAD,) if d == 1 else (NPAD, d)
    zshape = (RPT,) if d == 1 else (RPT, d)

    scratch = (
        [pltpu.VMEM((nb, BLK), i32)] * streams
        + [pltpu.VMEM(rshape, f32)] * (2 * streams)
        + ([pltpu.VMEM(zshape, f32), pltpu.VMEM_SHARED(tbl_shape, f32)]
           if via_spmem else [])
        + [pltpu.SemaphoreType.DMA] * (2 * streams)
    )

    @functools.partial(
        pl.kernel,
        out_type=jax.ShapeDtypeStruct(out_sh, f32),
        mesh=_MESH,
        scratch_types=scratch,
    )
    def k(*refs):
        it = list(refs)
        tbl_h = it.pop(0)
        ia_h = it.pop(0)
        ib_h = it.pop(0) if two else None
        out_h = it.pop(0)
        ia_v = it.pop(0)
        ib_v = it.pop(0) if two else None
        ra = [it.pop(0), it.pop(0)]
        rb = [it.pop(0), it.pop(0)] if two else [None, None]
        if via_spmem:
            zbuf = it.pop(0)
            tbl = it.pop(0)
        else:
            tbl = tbl_h
        sa = [it.pop(0), it.pop(0)]
        sb = [it.pop(0), it.pop(0)] if two else [None, None]
        c = lax.axis_index("c")
        s = lax.axis_index("s")
        w = s * NC + c
        if via_spmem:
            pltpu.sync_copy(tbl_h.at[pl.ds(s * RPT, RPT)], zbuf)
            pltpu.sync_copy(zbuf, tbl.at[pl.ds(s * RPT, RPT)])
        pltpu.sync_copy(ia_h.at[pl.ds(w * nb, nb)], ia_v)
        if two:
            pltpu.sync_copy(ib_h.at[pl.ds(w * nb, nb)], ib_v)
        if via_spmem:
            plsc.subcore_barrier()
        nvb = jnp.minimum(nb, jnp.maximum(nvb_tot - w * nb, 0))

        def issue(j, k2):
            pltpu.async_copy(tbl.at[ia_v.at[j]], ra[k2], sa[k2])
            if two:
                pltpu.async_copy(tbl.at[ib_v.at[j]], rb[k2], sb[k2])

        for k2 in range(2):
            @pl.when(k2 < nvb)
            def _():
                issue(k2, k2)

        def pair(j2, carry):
            for k2 in range(2):
                j = j2 * 2 + k2

                @pl.when(j < nvb)
                def _():
                    base = (w * nb + j) * BLK
                    pltpu.make_async_copy(tbl.at[ia_v.at[j]], ra[k2],
                                          sa[k2]).wait()
                    if two:
                        pltpu.make_async_copy(tbl.at[ib_v.at[j]], rb[k2],
                                              sb[k2]).wait()
                        if d == 1:
                            def addt(t, cc):
                                sl = pl.ds(t * LANES, LANES)
                                ra[k2][sl] = ra[k2][sl] + rb[k2][sl]
                                return cc
                            lax.fori_loop(0, BLK // LANES, addt, 0)
                        else:
                            nch = d // LANES
                            def addt(t, cc):
                                r = t // nch
                                kk = t % nch
                                sl = pl.ds(kk * LANES, LANES)
                                ra[k2][r, sl] = ra[k2][r, sl] + rb[k2][r, sl]
                                return cc
                            lax.fori_loop(0, BLK * nch, addt, 0)
                    pltpu.sync_copy(ra[k2], out_h.at[pl.ds(base, BLK)])

                    @pl.when(j + 2 < nvb)
                    def _():
                        issue(j + 2, k2)
            return carry

        lax.fori_loop(0, (nvb + 1) // 2, pair, 0)

    return k


def _tc_linear(a, w, b=None, act=None):
    m, kdim = a.shape
    nn = w.shape[1]
    bm = m if m <= 16384 else 4000
    assert m % bm == 0
    if b is None:
        b = jnp.zeros((nn,), f32)
    b2 = b.reshape(1, nn)

    def body(a_ref, w_ref, b_ref, o_ref):
        acc = jnp.dot(a_ref[...], w_ref[...], preferred_element_type=f32)
        acc = acc + b_ref[...]
        if act == "relu":
            acc = jnp.maximum(acc, 0.0)
        o_ref[...] = acc

    return pl.pallas_call(
        body,
        grid=(m // bm,),
        in_specs=[
            pl.BlockSpec((bm, kdim), lambda i: (i, 0)),
            pl.BlockSpec((kdim, nn), lambda i: (0, 0)),
            pl.BlockSpec((1, nn), lambda i: (0, 0)),
        ],
        out_specs=pl.BlockSpec((bm, nn), lambda i: (i, 0)),
        out_shape=jax.ShapeDtypeStruct((m, nn), f32),
    )(a, w, b2)


def _pad128(w):
    return jnp.pad(w, ((0, 128 - w.shape[0]), (0, 128 - w.shape[1])))


def _padvec128(b):
    return jnp.pad(b, (0, 128 - b.shape[0]))


def _padrows128(w):
    return jnp.pad(w, ((0, 128 - w.shape[0]), (0, 0)))


def kernel(x, edge_attr_, params, edge_index_, batch):
    src = edge_index_[0].astype(i32)
    dst = edge_index_[1].astype(i32)
    srcb = _pad_blocks(src, E)
    dstb = _pad_blocks(dst, E)
    ones_e = jnp.ones((E,), f32)
    zer1 = jnp.zeros((RPT,), f32)
    zer128 = jnp.zeros((QRPT, 128), f32)

    cntP = _scatter_kernel(E, 1, 2)(ones_e, srcb, dstb, zer1).reshape(NC, NPAD)
    cnt = cntP[0] + cntP[1]
    cnt_s = _gather_kernel(E, 1, 1)(cnt, srcb)
    cnt_d = _gather_kernel(E, 1, 1)(cnt, dstb)
    m_s = cnt_s != 1.0
    m_d = cnt_d != 1.0
    s_idx = jnp.where(m_s, src, DUMP)
    d_idx = jnp.where(m_d, dst, DUMP)
    s_idxb = _pad_blocks(s_idx, E)
    d_idxb = _pad_blocks(d_idx, E)
    dst_ns = _ns_idx(dst)
    s_ns = _ns_idx(src, m_s)
    d_ns = _ns_idx(dst, m_d)
    Dinv = 1.0 / (1.0 + m_s.astype(f32) + m_d.astype(f32))
    Binv = jnp.where((cnt != 1.0) & (cnt > 0.0), 1.0 / jnp.maximum(cnt, 1.0), 0.0)

    ea0 = _tc_linear(edge_attr_, _pad128(params["eattr_W"]),
                     _padvec128(params["eattr_b"]))
    xx0 = _tc_linear(x, params["node_W"], params["node_b"])

    convW = jnp.stack(params["conv_W"])
    convb = jnp.stack(params["conv_b"])
    hyperW = jnp.stack([_pad128(w) for w in params["hyper_W"]])
    hyperb = jnp.stack([_padvec128(b) for b in params["hyper_b"]])
    scoreW = jnp.stack([_padrows128(params["score_W"][0]),
                        _padrows128(params["score_W"][1]),
                        jnp.zeros((128, 1), f32)])
    scoreb = jnp.stack([params["score_b"][0], params["score_b"][1],
                        jnp.zeros((1,), f32)])
    notlast = jnp.array([1.0, 1.0, 0.0], f32)

    def layer(carry, inp):
        xx, ea, ew = carry
        cW, cb, hW, hb, sW, sb, nl = inp
        degP = _scatter_kernel(E, 1, 1)(ew, dstb, zer1).reshape(NC, NPAD)
        deg = 1.0 + (degP[0] + degP[1])[:N]
        dinv = lax.rsqrt(deg)
        h = _tc_linear(xx, cW)
        q = jnp.pad(dinv[:, None] * h, ((0, NPAD - N), (0, 0)))
        msg = _gather_kernel(E, 128, 1)(q, srcb) * ew[:, None]
        h_e = _tc_linear(ea, hW)
        SPf, PPf = _scatter_dual_kernel()(msg, dst_ns, h_e, s_ns, d_ns, zer128)
        SP = SPf.reshape(4, QPAD, 128)
        PP = PPf.reshape(4, QPAD, 128)
        S = jnp.concatenate([SP[q, :QN] for q in range(4)], axis=0)
        xxn = dinv[:, None] * S + (dinv**2)[:, None] * h + cb
        xxn = jnp.where(nl > 0.0, jnp.maximum(xxn, 0.0), xxn)
        P = jnp.concatenate([PP[q, :QN] for q in range(4)], axis=0)
        T = jnp.pad(Binv[:N, None] * P, ((0, NPAD - N), (0, 0)))
        gath = _gather_kernel(E, 128, 2)(T, s_idxb, d_idxb)
        out_n = Dinv[:, None] * (gath + h_e) + hb
        ean = jnp.where(nl > 0.0, jnp.maximum(out_n, 0.0), out_n)
        h_s = _tc_linear(ean, sW)[:, 0]
        PsP = _scatter_kernel(E, 1, 2)(h_s, s_idxb, d_idxb, zer1).reshape(NC, NPAD)
        Ts = Binv * (PsP[0] + PsP[1])
        gs = _gather_kernel(E, 1, 2)(Ts, s_idxb, d_idxb)
        score = jnp.tanh(Dinv * (gs + h_s) + sb[0])
        ewn = jnp.clip(score, 0.0, 1.0)
        return (xxn, ean, ewn), 0.0

    (xx, _, _), _ = lax.scan(
        layer, (xx0, ea0, ones_e),
        (convW, convb, hyperW, hyperb, scoreW, scoreb, notlast))
    return xx


# speedup vs baseline: 1.1198x; 1.1198x over previous
import functools

import jax
import jax.numpy as jnp
from jax import lax
from jax.experimental import pallas as pl
from jax.experimental.pallas import tpu as pltpu
from jax.experimental.pallas import tpu_sc as plsc

N = 10000
E = 320000
NC, NS, LANES = 2, 16, 16
NW = NC * NS
BLK = 128
NPAD = 10112
RPT = NPAD // NS
DUMP = N

f32 = jnp.float32
i32 = jnp.int32

_MESH = plsc.VectorSubcoreMesh(core_axis_name="c", subcore_axis_name="s")


def _nb(m):
    nb = -(-(m // NW) // BLK)
    return -(-nb // 8) * 8


def _pad_blocks(idx, m):
    nb = _nb(m)
    tot = NW * nb * BLK
    idx = jnp.pad(idx.astype(i32), (0, tot - m), constant_values=DUMP)
    return idx.reshape(NW * nb, BLK)


@functools.cache
def _scatter_kernel(m, d, streams):
    nb = _nb(m)
    nvb_tot = m // BLK
    two = streams == 2
    tbl_shape = (NPAD,) if d == 1 else (NPAD, d)
    vshape = (BLK,) if d == 1 else (BLK, d)
    out_sh = (NC * NPAD,) if d == 1 else (NC, NPAD, d)

    zshape = (RPT,) if d == 1 else (RPT, d)
    scratch = [pltpu.VMEM((nb, BLK), i32)] * streams + [
        pltpu.VMEM(vshape, f32),
        pltpu.VMEM(vshape, f32),
        pltpu.VMEM(zshape, f32),
        pltpu.VMEM_SHARED(tbl_shape, f32),
        pltpu.SemaphoreType.DMA,
        pltpu.SemaphoreType.DMA,
    ]

    @functools.partial(
        pl.kernel,
        out_type=jax.ShapeDtypeStruct(out_sh, f32),
        mesh=_MESH,
        scratch_types=scratch,
    )
    def k(*refs):
        if two:
            (vals_h, ia_h, ib_h, z_h, out_h, ia_v, ib_v,
             v0, v1, zbuf, tbl, sem0, sem1) = refs
        else:
            vals_h, ia_h, z_h, out_h, ia_v, v0, v1, zbuf, tbl, sem0, sem1 = refs
            ib_h = ib_v = None
        bufs = (v0, v1)
        sems = (sem0, sem1)
        c = lax.axis_index("c")
        s = lax.axis_index("s")
        w = s * NC + c
        pltpu.sync_copy(z_h, zbuf)
        pltpu.sync_copy(zbuf, tbl.at[pl.ds(s * RPT, RPT)])
        pltpu.sync_copy(ia_h.at[pl.ds(w * nb, nb)], ia_v)
        if two:
            pltpu.sync_copy(ib_h.at[pl.ds(w * nb, nb)], ib_v)
        plsc.subcore_barrier()
        nvb = jnp.minimum(nb, jnp.maximum(nvb_tot - w * nb, 0))

        for k2 in range(2):
            @pl.when(k2 < nvb)
            def _():
                base = (w * nb + k2) * BLK
                pltpu.async_copy(vals_h.at[pl.ds(base, BLK)], bufs[k2], sems[k2])

        def pairblk(j2, carry):
            for k2 in range(2):
                j = j2 * 2 + k2

                @pl.when(j < nvb)
                def _():
                    base = (w * nb + j) * BLK
                    pltpu.make_async_copy(vals_h.at[pl.ds(base, BLK)],
                                          bufs[k2], sems[k2]).wait()
                    pltpu.sync_copy(bufs[k2], tbl.at[ia_v.at[j]], add=True)
                    if two:
                        pltpu.sync_copy(bufs[k2], tbl.at[ib_v.at[j]], add=True)

                    @pl.when(j + 2 < nvb)
                    def _():
                        base2 = (w * nb + j + 2) * BLK
                        pltpu.async_copy(vals_h.at[pl.ds(base2, BLK)],
                                         bufs[k2], sems[k2])
            return carry

        lax.fori_loop(0, (nvb + 1) // 2, pairblk, 0)
        plsc.subcore_barrier()
        pltpu.sync_copy(tbl.at[pl.ds(s * RPT, RPT)], zbuf)
        if d == 1:
            pltpu.sync_copy(zbuf, out_h.at[pl.ds(c * NPAD + s * RPT, RPT)])
        else:
            pltpu.sync_copy(zbuf, out_h.at[c, pl.ds(s * RPT, RPT)])

    return k


QN = 2500
QPAD = 2560
QRPT = QPAD // NS
NDUMP = 32


@functools.cache
def _scatter_dual_kernel():
    d = 128
    cpw = E // NS
    nb = -(-(-(-cpw // BLK)) // 8) * 8
    nvb_tot = E // BLK

    scratch = [
        pltpu.VMEM((nb, BLK), i32),
        pltpu.VMEM((nb, BLK), i32),
        pltpu.VMEM((BLK, d), f32),
        pltpu.VMEM((BLK, d), f32),
        pltpu.VMEM((QRPT, d), f32),
        pltpu.VMEM_SHARED((QPAD, d), f32),
        pltpu.SemaphoreType.DMA,
        pltpu.SemaphoreType.DMA,
        pltpu.SemaphoreType.DMA,
        pltpu.SemaphoreType.DMA,
    ]

    @functools.partial(
        pl.kernel,
        out_type=(jax.ShapeDtypeStruct((4 * QPAD, d), f32),
                  jax.ShapeDtypeStruct((4 * QPAD, d), f32)),
        mesh=_MESH,
        scratch_types=scratch,
    )
    def k(va_h, ia_h, vb_h, ib1_h, ib2_h, z_h, outa_h, outb_h,
          i1_v, i2_v, v0, v1, zbuf, tbl, sem0, sem1, ssc0, ssc1):
        c = lax.axis_index("c")
        s = lax.axis_index("s")
        nvb = jnp.minimum(nb, jnp.maximum(nvb_tot - s * nb, 0))
        bufs = (v0, v1)
        sems = (sem0, sem1)
        ssems = (ssc0, ssc1)
        pltpu.sync_copy(z_h, zbuf)

        def sweep(vals_h, idx_list):
            for k2 in range(2):
                @pl.when(k2 < nvb)
                def _():
                    base = (s * nb + k2) * BLK
                    pltpu.async_copy(vals_h.at[pl.ds(base, BLK)],
                                     bufs[k2], sems[k2])

            def pair(j2, carry):
                for k2 in range(2):
                    j = j2 * 2 + k2

                    @pl.when(j < nvb)
                    def _():
                        base = (s * nb + j) * BLK
                        pltpu.make_async_copy(vals_h.at[pl.ds(base, BLK)],
                                              bufs[k2], sems[k2]).wait()
                        for iv in idx_list:
                            pltpu.async_copy(bufs[k2], tbl.at[iv.at[j]],
                                             ssems[k2], add=True)

                        @pl.when(j + 2 < nvb)
                        def _():
                            for iv in idx_list:
                                pltpu.make_async_copy(
                                    bufs[k2], tbl.at[iv.at[j]],
                                    ssems[k2]).wait()
                            base2 = (s * nb + j + 2) * BLK
                            pltpu.async_copy(vals_h.at[pl.ds(base2, BLK)],
                                             bufs[k2], sems[k2])

                        @pl.when(j + 2 >= nvb)
                        def _():
                            for iv in idx_list:
                                pltpu.make_async_copy(
                                    bufs[k2], tbl.at[iv.at[j]],
                                    ssems[k2]).wait()
                return carry

            lax.fori_loop(0, (nvb + 1) // 2, pair, 0)

        def run_pass(q, vals_h, idx_hs, idx_vs, out_h):
            pltpu.sync_copy(zbuf, tbl.at[pl.ds(s * QRPT, QRPT)])
            for ih, iv in zip(idx_hs, idx_vs):
                pltpu.sync_copy(ih.at[q, pl.ds(s * nb, nb)], iv)
            plsc.subcore_barrier()
            sweep(vals_h, idx_vs)
            plsc.subcore_barrier()
            pltpu.sync_copy(tbl.at[pl.ds(s * QRPT, QRPT)], zbuf)
            pltpu.sync_copy(zbuf, out_h.at[pl.ds(q * QPAD + s * QRPT, QRPT)])
            plsc.subcore_barrier()
            pltpu.sync_copy(z_h, zbuf)

        for p in range(2):
            run_pass(p * NC + c, va_h, [ia_h], [i1_v], outa_h)
        for p in range(2):
            run_pass(p * NC + c, vb_h, [ib1_h, ib2_h], [i1_v, i2_v], outb_h)

    return k


def _ns_idx(idx, valid=None):
    e_iota = jnp.arange(E, dtype=i32)
    dump = QN + (e_iota & (NDUMP - 1))
    outs = []
    for q in range(4):
        loc = idx - q * QN
        ok = (idx >= q * QN) & (idx < (q + 1) * QN)
        if valid is not None:
            ok = ok & valid
        outs.append(jnp.where(ok, loc, dump))
    return _pad_blocks_ns(jnp.stack(outs))


def _pad_blocks_ns(idx4):
    cpw = E // NS
    nb = -(-(-(-cpw // BLK)) // 8) * 8
    tot = NS * nb * BLK
    idx4 = jnp.pad(idx4, ((0, 0), (0, tot - E)), constant_values=QN)
    return idx4.reshape(4, NS * nb, BLK)


@functools.cache
def _gather_kernel(m, d, streams):
    nb = _nb(m)
    nvb_tot = m // BLK
    two = streams == 2
    via_spmem = d < 128
    rshape = (BLK,) if d == 1 else (BLK, d)
    out_sh = (m,) if d == 1 else (m, d)
    tbl_shape = (NPAD,) if d == 1 else (NPAD, d)
    zshape = (RPT,) if d == 1 else (RPT, d)

    scratch = (
        [pltpu.VMEM((nb, BLK), i32)] * streams
        + [pltpu.VMEM(rshape, f32)] * (2 * streams)
        + ([pltpu.VMEM(zshape, f32), pltpu.VMEM_SHARED(tbl_shape, f32)]
           if via_spmem else [])
        + [pltpu.SemaphoreType.DMA] * (2 * streams)
    )

    @functools.partial(
        pl.kernel,
        out_type=jax.ShapeDtypeStruct(out_sh, f32),
        mesh=_MESH,
        scratch_types=scratch,
    )
    def k(*refs):
        it = list(refs)
        tbl_h = it.pop(0)
        ia_h = it.pop(0)
        ib_h = it.pop(0) if two else None
        out_h = it.pop(0)
        ia_v = it.pop(0)
        ib_v = it.pop(0) if two else None
        ra = [it.pop(0), it.pop(0)]
        rb = [it.pop(0), it.pop(0)] if two else [None, None]
        if via_spmem:
            zbuf = it.pop(0)
            tbl = it.pop(0)
        else:
            tbl = tbl_h
        sa = [it.pop(0), it.pop(0)]
        sb = [it.pop(0), it.pop(0)] if two else [None, None]
        c = lax.axis_index("c")
        s = lax.axis_index("s")
        w = s * NC + c
        if via_spmem:
            pltpu.sync_copy(tbl_h.at[pl.ds(s * RPT, RPT)], zbuf)
            pltpu.sync_copy(zbuf, tbl.at[pl.ds(s * RPT, RPT)])
        pltpu.sync_copy(ia_h.at[pl.ds(w * nb, nb)], ia_v)
        if two:
            pltpu.sync_copy(ib_h.at[pl.ds(w * nb, nb)], ib_v)
        if via_spmem:
            plsc.subcore_barrier()
        nvb = jnp.minimum(nb, jnp.maximum(nvb_tot - w * nb, 0))

        def issue(j, k2):
            pltpu.async_copy(tbl.at[ia_v.at[j]], ra[k2], sa[k2])
            if two:
                pltpu.async_copy(tbl.at[ib_v.at[j]], rb[k2], sb[k2])

        for k2 in range(2):
            @pl.when(k2 < nvb)
            def _():
                issue(k2, k2)

        def pair(j2, carry):
            for k2 in range(2):
                j = j2 * 2 + k2

                @pl.when(j < nvb)
                def _():
                    base = (w * nb + j) * BLK
                    pltpu.make_async_copy(tbl.at[ia_v.at[j]], ra[k2],
                                          sa[k2]).wait()
                    if two:
                        pltpu.make_async_copy(tbl.at[ib_v.at[j]], rb[k2],
                                              sb[k2]).wait()
                        if d == 1:
                            def addt(t, cc):
                                sl = pl.ds(t * LANES, LANES)
                                ra[k2][sl] = ra[k2][sl] + rb[k2][sl]
                                return cc
                            lax.fori_loop(0, BLK // LANES, addt, 0)
                        else:
                            nch = d // LANES
                            def addt(r, cc):
                                for kk in range(nch):
                                    sl = pl.ds(kk * LANES, LANES)
                                    ra[k2][r, sl] = ra[k2][r, sl] + rb[k2][r, sl]
                                return cc
                            lax.fori_loop(0, BLK, addt, 0)
                    pltpu.sync_copy(ra[k2], out_h.at[pl.ds(base, BLK)])

                    @pl.when(j + 2 < nvb)
                    def _():
                        issue(j + 2, k2)
            return carry

        lax.fori_loop(0, (nvb + 1) // 2, pair, 0)

    return k


def _tc_linear(a, w, b=None, act=None):
    m, kdim = a.shape
    nn = w.shape[1]
    bm = m if m <= 16384 else 4000
    assert m % bm == 0
    if b is None:
        b = jnp.zeros((nn,), f32)
    b2 = b.reshape(1, nn)

    def body(a_ref, w_ref, b_ref, o_ref):
        acc = jnp.dot(a_ref[...], w_ref[...], preferred_element_type=f32)
        acc = acc + b_ref[...]
        if act == "relu":
            acc = jnp.maximum(acc, 0.0)
        o_ref[...] = acc

    return pl.pallas_call(
        body,
        grid=(m // bm,),
        in_specs=[
            pl.BlockSpec((bm, kdim), lambda i: (i, 0)),
            pl.BlockSpec((kdim, nn), lambda i: (0, 0)),
            pl.BlockSpec((1, nn), lambda i: (0, 0)),
        ],
        out_specs=pl.BlockSpec((bm, nn), lambda i: (i, 0)),
        out_shape=jax.ShapeDtypeStruct((m, nn), f32),
    )(a, w, b2)


def _pad128(w):
    return jnp.pad(w, ((0, 128 - w.shape[0]), (0, 128 - w.shape[1])))


def _padvec128(b):
    return jnp.pad(b, (0, 128 - b.shape[0]))


def _padrows128(w):
    return jnp.pad(w, ((0, 128 - w.shape[0]), (0, 0)))


def kernel(x, edge_attr_, params, edge_index_, batch):
    src = edge_index_[0].astype(i32)
    dst = edge_index_[1].astype(i32)
    srcb = _pad_blocks(src, E)
    dstb = _pad_blocks(dst, E)
    ones_e = jnp.ones((E,), f32)
    zer1 = jnp.zeros((RPT,), f32)
    zer128 = jnp.zeros((QRPT, 128), f32)

    cntP = _scatter_kernel(E, 1, 2)(ones_e, srcb, dstb, zer1).reshape(NC, NPAD)
    cnt = cntP[0] + cntP[1]
    cnt_s = _gather_kernel(E, 1, 1)(cnt, srcb)
    cnt_d = _gather_kernel(E, 1, 1)(cnt, dstb)
    m_s = cnt_s != 1.0
    m_d = cnt_d != 1.0
    s_idx = jnp.where(m_s, src, DUMP)
    d_idx = jnp.where(m_d, dst, DUMP)
    s_idxb = _pad_blocks(s_idx, E)
    d_idxb = _pad_blocks(d_idx, E)
    dst_ns = _ns_idx(dst)
    s_ns = _ns_idx(src, m_s)
    d_ns = _ns_idx(dst, m_d)
    Dinv = 1.0 / (1.0 + m_s.astype(f32) + m_d.astype(f32))
    Binv = jnp.where((cnt != 1.0) & (cnt > 0.0), 1.0 / jnp.maximum(cnt, 1.0), 0.0)

    ea0 = _tc_linear(edge_attr_, _pad128(params["eattr_W"]),
                     _padvec128(params["eattr_b"]))
    xx0 = _tc_linear(x, params["node_W"], params["node_b"])

    convW = jnp.stack(params["conv_W"])
    convb = jnp.stack(params["conv_b"])
    hyperW = jnp.stack([_pad128(w) for w in params["hyper_W"]])
    hyperb = jnp.stack([_padvec128(b) for b in params["hyper_b"]])
    scoreW = jnp.stack([_padrows128(params["score_W"][0]),
                        _padrows128(params["score_W"][1]),
                        jnp.zeros((128, 1), f32)])
    scoreb = jnp.stack([params["score_b"][0], params["score_b"][1],
                        jnp.zeros((1,), f32)])
    notlast = jnp.array([1.0, 1.0, 0.0], f32)

    def layer(carry, inp):
        xx, ea, ew = carry
        cW, cb, hW, hb, sW, sb, nl = inp
        degP = _scatter_kernel(E, 1, 1)(ew, dstb, zer1).reshape(NC, NPAD)
        deg = 1.0 + (degP[0] + degP[1])[:N]
        dinv = lax.rsqrt(deg)
        h = _tc_linear(xx, cW)
        q = jnp.pad(dinv[:, None] * h, ((0, NPAD - N), (0, 0)))
        msg = _gather_kernel(E, 128, 1)(q, srcb) * ew[:, None]
        h_e = _tc_linear(ea, hW)
        SPf, PPf = _scatter_dual_kernel()(msg, dst_ns, h_e, s_ns, d_ns, zer128)
        SP = SPf.reshape(4, QPAD, 128)
        PP = PPf.reshape(4, QPAD, 128)
        S = jnp.concatenate([SP[q, :QN] for q in range(4)], axis=0)
        xxn = dinv[:, None] * S + (dinv**2)[:, None] * h + cb
        xxn = jnp.where(nl > 0.0, jnp.maximum(xxn, 0.0), xxn)
        P = jnp.concatenate([PP[q, :QN] for q in range(4)], axis=0)
        T = jnp.pad(Binv[:N, None] * P, ((0, NPAD - N), (0, 0)))
        gath = _gather_kernel(E, 128, 2)(T, s_idxb, d_idxb)
        out_n = Dinv[:, None] * (gath + h_e) + hb
        ean = jnp.where(nl > 0.0, jnp.maximum(out_n, 0.0), out_n)
        h_s = _tc_linear(ean, sW)[:, 0]
        PsP = _scatter_kernel(E, 1, 2)(h_s, s_idxb, d_idxb, zer1).reshape(NC, NPAD)
        Ts = Binv * (PsP[0] + PsP[1])
        gs = _gather_kernel(E, 1, 2)(Ts, s_idxb, d_idxb)
        score = jnp.tanh(Dinv * (gs + h_s) + sb[0])
        ewn = jnp.clip(score, 0.0, 1.0)
        return (xxn, ean, ewn), 0.0

    (xx, _, _), _ = lax.scan(
        layer, (xx0, ea0, ones_e),
        (convW, convb, hyperW, hyperb, scoreW, scoreb, notlast))
    return xx


# speedup vs baseline: 1.1207x; 1.0008x over previous
import functools

import jax
import jax.numpy as jnp
from jax import lax
from jax.experimental import pallas as pl
from jax.experimental.pallas import tpu as pltpu
from jax.experimental.pallas import tpu_sc as plsc

N = 10000
E = 320000
NC, NS, LANES = 2, 16, 16
NW = NC * NS
BLK = 128
NPAD = 10112
RPT = NPAD // NS
DUMP = N

f32 = jnp.float32
i32 = jnp.int32

_MESH = plsc.VectorSubcoreMesh(core_axis_name="c", subcore_axis_name="s")


def _nb(m):
    nb = -(-(m // NW) // BLK)
    return -(-nb // 8) * 8


def _pad_blocks(idx, m):
    nb = _nb(m)
    tot = NW * nb * BLK
    idx = jnp.pad(idx.astype(i32), (0, tot - m), constant_values=DUMP)
    return idx.reshape(NW * nb, BLK)


@functools.cache
def _scatter_kernel(m, d, streams):
    nb = _nb(m)
    nvb_tot = m // BLK
    two = streams == 2
    tbl_shape = (NPAD,) if d == 1 else (NPAD, d)
    vshape = (BLK,) if d == 1 else (BLK, d)
    out_sh = (NC * NPAD,) if d == 1 else (NC, NPAD, d)

    zshape = (RPT,) if d == 1 else (RPT, d)
    scratch = [pltpu.VMEM((nb, BLK), i32)] * streams + [
        pltpu.VMEM(vshape, f32),
        pltpu.VMEM(vshape, f32),
        pltpu.VMEM(zshape, f32),
        pltpu.VMEM_SHARED(tbl_shape, f32),
        pltpu.SemaphoreType.DMA,
        pltpu.SemaphoreType.DMA,
    ]

    @functools.partial(
        pl.kernel,
        out_type=jax.ShapeDtypeStruct(out_sh, f32),
        mesh=_MESH,
        scratch_types=scratch,
    )
    def k(*refs):
        if two:
            (vals_h, ia_h, ib_h, z_h, out_h, ia_v, ib_v,
             v0, v1, zbuf, tbl, sem0, sem1) = refs
        else:
            vals_h, ia_h, z_h, out_h, ia_v, v0, v1, zbuf, tbl, sem0, sem1 = refs
            ib_h = ib_v = None
        bufs = (v0, v1)
        sems = (sem0, sem1)
        c = lax.axis_index("c")
        s = lax.axis_index("s")
        w = s * NC + c
        pltpu.sync_copy(z_h, zbuf)
        pltpu.sync_copy(zbuf, tbl.at[pl.ds(s * RPT, RPT)])
        pltpu.sync_copy(ia_h.at[pl.ds(w * nb, nb)], ia_v)
        if two:
            pltpu.sync_copy(ib_h.at[pl.ds(w * nb, nb)], ib_v)
        plsc.subcore_barrier()
        nvb = jnp.minimum(nb, jnp.maximum(nvb_tot - w * nb, 0))

        for k2 in range(2):
            @pl.when(k2 < nvb)
            def _():
                base = (w * nb + k2) * BLK
                pltpu.async_copy(vals_h.at[pl.ds(base, BLK)], bufs[k2], sems[k2])

        def pairblk(j2, carry):
            for k2 in range(2):
                j = j2 * 2 + k2

                @pl.when(j < nvb)
                def _():
                    base = (w * nb + j) * BLK
                    pltpu.make_async_copy(vals_h.at[pl.ds(base, BLK)],
                                          bufs[k2], sems[k2]).wait()
                    pltpu.sync_copy(bufs[k2], tbl.at[ia_v.at[j]], add=True)
                    if two:
                        pltpu.sync_copy(bufs[k2], tbl.at[ib_v.at[j]], add=True)

                    @pl.when(j + 2 < nvb)
                    def _():
                        base2 = (w * nb + j + 2) * BLK
                        pltpu.async_copy(vals_h.at[pl.ds(base2, BLK)],
                                         bufs[k2], sems[k2])
            return carry

        lax.fori_loop(0, (nvb + 1) // 2, pairblk, 0)
        plsc.subcore_barrier()
        pltpu.sync_copy(tbl.at[pl.ds(s * RPT, RPT)], zbuf)
        if d == 1:
            pltpu.sync_copy(zbuf, out_h.at[pl.ds(c * NPAD + s * RPT, RPT)])
        else:
            pltpu.sync_copy(zbuf, out_h.at[c, pl.ds(s * RPT, RPT)])

    return k


QN = 2500
QPAD = 2560
QRPT = QPAD // NS
NDUMP = 32


@functools.cache
def _scatter_dual_kernel():
    d = 128
    cpw = E // NS
    nb = -(-(-(-cpw // BLK)) // 8) * 8
    nvb_tot = E // BLK

    scratch = [
        pltpu.VMEM((nb, BLK), i32),
        pltpu.VMEM((nb, BLK), i32),
        pltpu.VMEM((BLK, d), f32),
        pltpu.VMEM((BLK, d), f32),
        pltpu.VMEM((QRPT, d), f32),
        pltpu.VMEM_SHARED((QPAD, d), f32),
        pltpu.SemaphoreType.DMA,
        pltpu.SemaphoreType.DMA,
        pltpu.SemaphoreType.DMA,
        pltpu.SemaphoreType.DMA,
    ]

    @functools.partial(
        pl.kernel,
        out_type=(jax.ShapeDtypeStruct((4 * QPAD, d), f32),
                  jax.ShapeDtypeStruct((4 * QPAD, d), f32)),
        mesh=_MESH,
        scratch_types=scratch,
    )
    def k(va_h, ia_h, vb_h, ib1_h, ib2_h, z_h, outa_h, outb_h,
          i1_v, i2_v, v0, v1, zbuf, tbl, sem0, sem1, ssc0, ssc1):
        c = lax.axis_index("c")
        s = lax.axis_index("s")
        nvb = jnp.minimum(nb, jnp.maximum(nvb_tot - s * nb, 0))
        bufs = (v0, v1)
        sems = (sem0, sem1)
        ssems = (ssc0, ssc1)
        pltpu.sync_copy(z_h, zbuf)

        def sweep(vals_h, idx_list):
            for k2 in range(2):
                @pl.when(k2 < nvb)
                def _():
                    base = (s * nb + k2) * BLK
                    pltpu.async_copy(vals_h.at[pl.ds(base, BLK)],
                                     bufs[k2], sems[k2])

            def pair(j2, carry):
                for k2 in range(2):
                    j = j2 * 2 + k2

                    @pl.when(j < nvb)
                    def _():
                        base = (s * nb + j) * BLK
                        pltpu.make_async_copy(vals_h.at[pl.ds(base, BLK)],
                                              bufs[k2], sems[k2]).wait()
                        for iv in idx_list:
                            pltpu.async_copy(bufs[k2], tbl.at[iv.at[j]],
                                             ssems[k2], add=True)

                        @pl.when(j + 2 < nvb)
                        def _():
                            for iv in idx_list:
                                pltpu.make_async_copy(
                                    bufs[k2], tbl.at[iv.at[j]],
                                    ssems[k2]).wait()
                            base2 = (s * nb + j + 2) * BLK
                            pltpu.async_copy(vals_h.at[pl.ds(base2, BLK)],
                                             bufs[k2], sems[k2])

                        @pl.when(j + 2 >= nvb)
                        def _():
                            for iv in idx_list:
                                pltpu.make_async_copy(
                                    bufs[k2], tbl.at[iv.at[j]],
                                    ssems[k2]).wait()
                return carry

            lax.fori_loop(0, (nvb + 1) // 2, pair, 0)

        def run_pass(q, vals_h, idx_hs, idx_vs, out_h):
            pltpu.sync_copy(zbuf, tbl.at[pl.ds(s * QRPT, QRPT)])
            for ih, iv in zip(idx_hs, idx_vs):
                pltpu.sync_copy(ih.at[pl.ds(q * (NS * nb) + s * nb, nb)], iv)
            plsc.subcore_barrier()
            sweep(vals_h, idx_vs)
            plsc.subcore_barrier()
            pltpu.sync_copy(tbl.at[pl.ds(s * QRPT, QRPT)], zbuf)
            pltpu.sync_copy(zbuf, out_h.at[pl.ds(q * QPAD + s * QRPT, QRPT)])
            plsc.subcore_barrier()
            pltpu.sync_copy(z_h, zbuf)

        for p in range(2):
            run_pass(p * NC + c, va_h, [ia_h], [i1_v], outa_h)
        for p in range(2):
            run_pass(p * NC + c, vb_h, [ib1_h, ib2_h], [i1_v, i2_v], outb_h)

    return k


def _ns_idx(idx, valid=None):
    e_iota = jnp.arange(E, dtype=i32)
    dump = QN + (e_iota & (NDUMP - 1))
    outs = []
    for q in range(4):
        loc = idx - q * QN
        ok = (idx >= q * QN) & (idx < (q + 1) * QN)
        if valid is not None:
            ok = ok & valid
        outs.append(jnp.where(ok, loc, dump))
    return _pad_blocks_ns(jnp.stack(outs))


def _pad_blocks_ns(idx4):
    cpw = E // NS
    nb = -(-(-(-cpw // BLK)) // 8) * 8
    tot = NS * nb * BLK
    idx4 = jnp.pad(idx4, ((0, 0), (0, tot - E)), constant_values=QN)
    return idx4.reshape(4 * NS * nb, BLK)


@functools.cache
def _gather_kernel(m, d, streams):
    nb = _nb(m)
    nvb_tot = m // BLK
    two = streams == 2
    via_spmem = d < 128
    rshape = (BLK,) if d == 1 else (BLK, d)
    out_sh = (m,) if d == 1 else (m, d)
    tbl_shape = (NPAD,) if d == 1 else (NPAD, d)
    zshape = (RPT,) if d == 1 else (RPT, d)

    scratch = (
        [pltpu.VMEM((nb, BLK), i32)] * streams
        + [pltpu.VMEM(rshape, f32)] * (2 * streams)
        + ([pltpu.VMEM(zshape, f32), pltpu.VMEM_SHARED(tbl_shape, f32)]
           if via_spmem else [])
        + [pltpu.SemaphoreType.DMA] * (2 * streams)
    )

    @functools.partial(
        pl.kernel,
        out_type=jax.ShapeDtypeStruct(out_sh, f32),
        mesh=_MESH,
        scratch_types=scratch,
    )
    def k(*refs):
        it = list(refs)
        tbl_h = it.pop(0)
        ia_h = it.pop(0)
        ib_h = it.pop(0) if two else None
        out_h = it.pop(0)
        ia_v = it.pop(0)
        ib_v = it.pop(0) if two else None
        ra = [it.pop(0), it.pop(0)]
        rb = [it.pop(0), it.pop(0)] if two else [None, None]
        if via_spmem:
            zbuf = it.pop(0)
            tbl = it.pop(0)
        else:
            tbl = tbl_h
        sa = [it.pop(0), it.pop(0)]
        sb = [it.pop(0), it.pop(0)] if two else [None, None]
        c = lax.axis_index("c")
        s = lax.axis_index("s")
        w = s * NC + c
        if via_spmem:
            pltpu.sync_copy(tbl_h.at[pl.ds(s * RPT, RPT)], zbuf)
            pltpu.sync_copy(zbuf, tbl.at[pl.ds(s * RPT, RPT)])
        pltpu.sync_copy(ia_h.at[pl.ds(w * nb, nb)], ia_v)
        if two:
            pltpu.sync_copy(ib_h.at[pl.ds(w * nb, nb)], ib_v)
        if via_spmem:
            plsc.subcore_barrier()
        nvb = jnp.minimum(nb, jnp.maximum(nvb_tot - w * nb, 0))

        def issue(j, k2):
            pltpu.async_copy(tbl.at[ia_v.at[j]], ra[k2], sa[k2])
            if two:
                pltpu.async_copy(tbl.at[ib_v.at[j]], rb[k2], sb[k2])

        for k2 in range(2):
            @pl.when(k2 < nvb)
            def _():
                issue(k2, k2)

        def pair(j2, carry):
            for k2 in range(2):
                j = j2 * 2 + k2

                @pl.when(j < nvb)
                def _():
                    base = (w * nb + j) * BLK
                    pltpu.make_async_copy(tbl.at[ia_v.at[j]], ra[k2],
                                          sa[k2]).wait()
                    if two:
                        pltpu.make_async_copy(tbl.at[ib_v.at[j]], rb[k2],
                                              sb[k2]).wait()
                        if d == 1:
                            def addt(t, cc):
                                sl = pl.ds(t * LANES, LANES)
                                ra[k2][sl] = ra[k2][sl] + rb[k2][sl]
                                return cc
                            lax.fori_loop(0, BLK // LANES, addt, 0)
                        else:
                            nch = d // LANES
                            def addt(r, cc):
                                for kk in range(nch):
                                    sl = pl.ds(kk * LANES, LANES)
                                    ra[k2][r, sl] = ra[k2][r, sl] + rb[k2][r, sl]
                                return cc
                            lax.fori_loop(0, BLK, addt, 0)
                    pltpu.sync_copy(ra[k2], out_h.at[pl.ds(base, BLK)])

                    @pl.when(j + 2 < nvb)
                    def _():
                        issue(j + 2, k2)
            return carry

        lax.fori_loop(0, (nvb + 1) // 2, pair, 0)

    return k


def _tc_linear(a, w, b=None, act=None):
    m, kdim = a.shape
    nn = w.shape[1]
    bm = m if m <= 16384 else 4000
    assert m % bm == 0
    if b is None:
        b = jnp.zeros((nn,), f32)
    b2 = b.reshape(1, nn)

    def body(a_ref, w_ref, b_ref, o_ref):
        acc = jnp.dot(a_ref[...], w_ref[...], preferred_element_type=f32)
        acc = acc + b_ref[...]
        if act == "relu":
            acc = jnp.maximum(acc, 0.0)
        o_ref[...] = acc

    return pl.pallas_call(
        body,
        grid=(m // bm,),
        in_specs=[
            pl.BlockSpec((bm, kdim), lambda i: (i, 0)),
            pl.BlockSpec((kdim, nn), lambda i: (0, 0)),
            pl.BlockSpec((1, nn), lambda i: (0, 0)),
        ],
        out_specs=pl.BlockSpec((bm, nn), lambda i: (i, 0)),
        out_shape=jax.ShapeDtypeStruct((m, nn), f32),
    )(a, w, b2)


def _pad128(w):
    return jnp.pad(w, ((0, 128 - w.shape[0]), (0, 128 - w.shape[1])))


def _padvec128(b):
    return jnp.pad(b, (0, 128 - b.shape[0]))


def _padrows128(w):
    return jnp.pad(w, ((0, 128 - w.shape[0]), (0, 0)))


def kernel(x, edge_attr_, params, edge_index_, batch):
    src = edge_index_[0].astype(i32)
    dst = edge_index_[1].astype(i32)
    srcb = _pad_blocks(src, E)
    dstb = _pad_blocks(dst, E)
    ones_e = jnp.ones((E,), f32)
    zer1 = jnp.zeros((RPT,), f32)
    zer128 = jnp.zeros((QRPT, 128), f32)

    cntP = _scatter_kernel(E, 1, 2)(ones_e, srcb, dstb, zer1).reshape(NC, NPAD)
    cnt = cntP[0] + cntP[1]
    cnt_s = _gather_kernel(E, 1, 1)(cnt, srcb)
    cnt_d = _gather_kernel(E, 1, 1)(cnt, dstb)
    m_s = cnt_s != 1.0
    m_d = cnt_d != 1.0
    s_idx = jnp.where(m_s, src, DUMP)
    d_idx = jnp.where(m_d, dst, DUMP)
    s_idxb = _pad_blocks(s_idx, E)
    d_idxb = _pad_blocks(d_idx, E)
    dst_ns = _ns_idx(dst)
    s_ns = _ns_idx(src, m_s)
    d_ns = _ns_idx(dst, m_d)
    Dinv = 1.0 / (1.0 + m_s.astype(f32) + m_d.astype(f32))
    Binv = jnp.where((cnt != 1.0) & (cnt > 0.0), 1.0 / jnp.maximum(cnt, 1.0), 0.0)

    ea0 = _tc_linear(edge_attr_, _pad128(params["eattr_W"]),
                     _padvec128(params["eattr_b"]))
    xx0 = _tc_linear(x, params["node_W"], params["node_b"])

    convW = jnp.stack(params["conv_W"])
    convb = jnp.stack(params["conv_b"])
    hyperW = jnp.stack([_pad128(w) for w in params["hyper_W"]])
    hyperb = jnp.stack([_padvec128(b) for b in params["hyper_b"]])
    scoreW = jnp.stack([_padrows128(params["score_W"][0]),
                        _padrows128(params["score_W"][1]),
                        jnp.zeros((128, 1), f32)])
    scoreb = jnp.stack([params["score_b"][0], params["score_b"][1],
                        jnp.zeros((1,), f32)])
    notlast = jnp.array([1.0, 1.0, 0.0], f32)

    def layer(carry, inp):
        xx, ea, ew = carry
        cW, cb, hW, hb, sW, sb, nl = inp
        degP = _scatter_kernel(E, 1, 1)(ew, dstb, zer1).reshape(NC, NPAD)
        deg = 1.0 + (degP[0] + degP[1])[:N]
        dinv = lax.rsqrt(deg)
        h = _tc_linear(xx, cW)
        q = jnp.pad(dinv[:, None] * h, ((0, NPAD - N), (0, 0)))
        msg = _gather_kernel(E, 128, 1)(q, srcb) * ew[:, None]
        h_e = _tc_linear(ea, hW)
        SPf, PPf = _scatter_dual_kernel()(msg, dst_ns, h_e, s_ns, d_ns, zer128)
        SP = SPf.reshape(4, QPAD, 128)
        PP = PPf.reshape(4, QPAD, 128)
        S = jnp.concatenate([SP[q, :QN] for q in range(4)], axis=0)
        xxn = dinv[:, None] * S + (dinv**2)[:, None] * h + cb
        xxn = jnp.where(nl > 0.0, jnp.maximum(xxn, 0.0), xxn)
        P = jnp.concatenate([PP[q, :QN] for q in range(4)], axis=0)
        T = jnp.pad(Binv[:N, None] * P, ((0, NPAD - N), (0, 0)))
        gath = _gather_kernel(E, 128, 2)(T, s_idxb, d_idxb)
        out_n = Dinv[:, None] * (gath + h_e) + hb
        ean = jnp.where(nl > 0.0, jnp.maximum(out_n, 0.0), out_n)
        h_s = _tc_linear(ean, sW)[:, 0]
        PsP = _scatter_kernel(E, 1, 2)(h_s, s_idxb, d_idxb, zer1).reshape(NC, NPAD)
        Ts = Binv * (PsP[0] + PsP[1])
        gs = _gather_kernel(E, 1, 2)(Ts, s_idxb, d_idxb)
        score = jnp.tanh(Dinv * (gs + h_s) + sb[0])
        ewn = jnp.clip(score, 0.0, 1.0)
        return (xxn, ean, ewn), 0.0

    (xx, _, _), _ = lax.scan(
        layer, (xx0, ea0, ones_e),
        (convW, convb, hyperW, hyperb, scoreW, scoreb, notlast))
    return xx
